# baseline (speedup 1.0000x reference)
import functools

import jax
import jax.numpy as jnp
from jax import lax
from jax.experimental import pallas as pl
from jax.experimental.pallas import tpu as pltpu
from jax.experimental.pallas import tpu_sc as plsc

_B = 16384
_D = 64
_NF = 5
_CH = 128

_info = plsc.get_sparse_core_info()
_NC = _info.num_cores
_NS = _info.num_subcores
_NW = _NC * _NS
_BPW = _B // _NW
_NCH = _BPW // _CH

_NBUF = 3


def _build():
    mesh = plsc.VectorSubcoreMesh(core_axis_name="c", subcore_axis_name="s")

    @functools.partial(
        pl.kernel,
        mesh=mesh,
        out_type=jax.ShapeDtypeStruct((_B, _NF * _D), jnp.float32),
        scratch_types=[
            pltpu.VMEM((_NF, _NCH, _CH), jnp.int32),
            [pltpu.VMEM((_BPW, _D), jnp.float32) for _ in range(_NBUF)],
            pltpu.SemaphoreType.DMA,
            [pltpu.SemaphoreType.DMA for _ in range(_NBUF)],
            [pltpu.SemaphoreType.DMA for _ in range(_NBUF)],
        ],
        compiler_params=pltpu.CompilerParams(use_tc_tiling_on_sc=False),
    )
    def sc_kernel(idx_stk, t_tab, f_tab, st_tab, so_tab, y_tab,
                  out, idx_v, bufs, isem, gsems, ssems):
        wid = lax.axis_index("s") * _NC + lax.axis_index("c")
        base = wid * _BPW
        tables = (t_tab, f_tab, st_tab, so_tab, y_tab)

        pltpu.async_copy(idx_stk.at[wid], idx_v, isem).wait()

        gathers = {}
        scatters = {}

        def start_gathers(fi):
            slot = fi % _NBUF
            gathers[fi] = [
                pltpu.async_copy(
                    tables[fi].at[idx_v.at[fi, c]],
                    bufs[slot].at[pl.ds(c * _CH, _CH)],
                    gsems[slot])
                for c in range(_NCH)
            ]

        for fi in range(_NBUF):
            start_gathers(fi)

        for fi in range(_NF):
            slot = fi % _NBUF
            for g in gathers[fi]:
                g.wait()
            scatters[fi] = pltpu.async_copy(
                bufs[slot],
                out.at[pl.ds(base, _BPW), pl.ds(fi * _D, _D)],
                ssems[slot])
            nxt = fi + _NBUF
            if nxt < _NF:
                scatters[fi].wait()
                start_gathers(nxt)

        for fi in range(_NF):
            if fi + _NBUF >= _NF:
                scatters[fi].wait()

    return sc_kernel


_sc_kernel = _build()


@jax.jit
def kernel(title_idx, format_idx, studio_idx, source_idx, year_idx,
           title_table, format_table, studio_table, source_table, year_table):
    idx_stk = jnp.stack(
        [title_idx, format_idx, studio_idx, source_idx, year_idx]
    ).reshape(_NF, _NW, _BPW).transpose(1, 0, 2).reshape(_NW, _NF, _NCH, _CH)
    return _sc_kernel(idx_stk, title_table, format_table, studio_table,
                      source_table, year_table)
